# baseline (speedup 1.0000x reference)
import functools

import jax
import jax.numpy as jnp
from jax import lax
from jax.experimental import pallas as pl
from jax.experimental.pallas import tpu as pltpu
from jax.experimental.pallas import tpu_sc as plsc

B = 1024
S = 77
SP = 80
H = SP // 2
D = 768
NC, NS = 2, 16
NW = NC * NS
SEQ_PER_W = B // NW
BB = 8

_mesh = plsc.VectorSubcoreMesh(core_axis_name="c", subcore_axis_name="s")


@functools.partial(
    pl.kernel,
    mesh=_mesh,
    out_type=(
        jax.ShapeDtypeStruct((B, SP, D), jnp.float32),
        jax.ShapeDtypeStruct((SP, D), jnp.float32),
    ),
    scratch_types=[
        pltpu.VMEM((SEQ_PER_W * SP,), jnp.int32),
        pltpu.VMEM((SP,), jnp.int32),
        pltpu.VMEM((SP, D), jnp.float32),
        pltpu.VMEM((H, D), jnp.float32),
        pltpu.VMEM((H, D), jnp.float32),
        pltpu.SemaphoreType.DMA,
        pltpu.SemaphoreType.DMA,
        pltpu.SemaphoreType.DMA,
        pltpu.SemaphoreType.DMA,
    ],
)
def _gather_kernel(ids_hbm, pids_hbm, tok_hbm, pos_hbm, out_hbm, pos_out_hbm,
                   idx_v, pidx_v, pos_v, buf0, buf1,
                   gsem0, gsem1, osem0, osem1):
    wid = lax.axis_index("s") * NC + lax.axis_index("c")

    pltpu.sync_copy(ids_hbm.at[pl.ds(wid * (SEQ_PER_W * SP), SEQ_PER_W * SP)],
                    idx_v)
    pltpu.sync_copy(pids_hbm, pidx_v)
    pltpu.async_copy(pos_hbm.at[pidx_v], pos_v, gsem0).wait()

    @pl.when(wid == 0)
    def _():
        pltpu.sync_copy(pos_v, pos_out_hbm)

    def seq_body(q, _):
        batch = wid * SEQ_PER_W + q

        @pl.when(q > 0)
        def _():
            pltpu.make_async_copy(
                buf0, out_hbm.at[batch - 1, pl.ds(0, H)], osem0).wait()
        g0 = pltpu.async_copy(tok_hbm.at[idx_v.at[pl.ds(q * SP, H)]],
                              buf0, gsem0)

        @pl.when(q > 0)
        def _():
            pltpu.make_async_copy(
                buf1, out_hbm.at[batch - 1, pl.ds(H, H)], osem1).wait()
        g1 = pltpu.async_copy(tok_hbm.at[idx_v.at[pl.ds(q * SP + H, H)]],
                              buf1, gsem1)

        g0.wait()
        pltpu.async_copy(buf0, out_hbm.at[batch, pl.ds(0, H)], osem0)
        g1.wait()
        pltpu.async_copy(buf1, out_hbm.at[batch, pl.ds(H, H)], osem1)
        return 0

    lax.fori_loop(0, SEQ_PER_W, seq_body, 0)

    last = wid * SEQ_PER_W + SEQ_PER_W - 1
    pltpu.make_async_copy(buf0, out_hbm.at[last, pl.ds(0, H)], osem0).wait()
    pltpu.make_async_copy(buf1, out_hbm.at[last, pl.ds(H, H)], osem1).wait()


def _add_body(g_ref, p_ref, o_ref):
    o_ref[...] = g_ref[:, :S, :] + p_ref[:, :S, :]


_add_kernel = pl.pallas_call(
    _add_body,
    grid=(B // BB,),
    in_specs=[
        pl.BlockSpec((BB, SP, D), lambda i: (i, 0, 0)),
        pl.BlockSpec((1, SP, D), lambda i: (0, 0, 0)),
    ],
    out_specs=pl.BlockSpec((BB, S, D), lambda i: (i, 0, 0)),
    out_shape=jax.ShapeDtypeStruct((B, S, D), jnp.float32),
)


def kernel(input_ids, position_ids, token_table, pos_table):
    ids = input_ids.astype(jnp.int32).reshape(B, S)
    ids_pad = jnp.pad(ids, ((0, 0), (0, SP - S))).reshape(-1)
    pids = jnp.pad(position_ids.astype(jnp.int32).reshape(-1), (0, SP - S))
    gathered, pos_eff = _gather_kernel(ids_pad, pids, token_table, pos_table)
    return _add_kernel(gathered, pos_eff[None])


# speedup vs baseline: 1.0007x; 1.0007x over previous
import functools

import jax
import jax.numpy as jnp
from jax import lax
from jax.experimental import pallas as pl
from jax.experimental.pallas import tpu as pltpu
from jax.experimental.pallas import tpu_sc as plsc

B = 1024
S = 77
SP = 80
H = SP // 2
D = 768
NC, NS = 2, 16
NW = NC * NS
SEQ_PER_W = B // NW
BB = 8

_mesh = plsc.VectorSubcoreMesh(core_axis_name="c", subcore_axis_name="s")


@functools.partial(
    pl.kernel,
    mesh=_mesh,
    out_type=(
        jax.ShapeDtypeStruct((B * SP, D), jnp.float32),
        jax.ShapeDtypeStruct((SP, D), jnp.float32),
    ),
    scratch_types=[
        pltpu.VMEM((SEQ_PER_W * SP,), jnp.int32),
        pltpu.VMEM((SP,), jnp.int32),
        pltpu.VMEM((SP, D), jnp.float32),
        pltpu.VMEM((H, D), jnp.float32),
        pltpu.VMEM((H, D), jnp.float32),
        pltpu.SemaphoreType.DMA,
        pltpu.SemaphoreType.DMA,
        pltpu.SemaphoreType.DMA,
        pltpu.SemaphoreType.DMA,
    ],
)
def _gather_kernel(ids_hbm, pids_hbm, tok_hbm, pos_hbm, out_hbm, pos_out_hbm,
                   idx_v, pidx_v, pos_v, buf0, buf1,
                   gsem0, gsem1, osem0, osem1):
    wid = lax.axis_index("s") * NC + lax.axis_index("c")
    wbase = wid * (SEQ_PER_W * SP)

    pltpu.sync_copy(ids_hbm.at[pl.ds(wbase, SEQ_PER_W * SP)], idx_v)
    pltpu.sync_copy(pids_hbm, pidx_v)
    pltpu.async_copy(pos_hbm.at[pidx_v], pos_v, gsem0).wait()

    @pl.when(wid == 0)
    def _():
        pltpu.sync_copy(pos_v, pos_out_hbm)

    def seq_body(q, _):
        base = wbase + q * SP

        @pl.when(q > 0)
        def _():
            pltpu.make_async_copy(
                buf0, out_hbm.at[pl.ds(base - SP, H)], osem0).wait()
        g0 = pltpu.async_copy(tok_hbm.at[idx_v.at[pl.ds(q * SP, H)]],
                              buf0, gsem0)

        @pl.when(q > 0)
        def _():
            pltpu.make_async_copy(
                buf1, out_hbm.at[pl.ds(base - SP + H, H)], osem1).wait()
        g1 = pltpu.async_copy(tok_hbm.at[idx_v.at[pl.ds(q * SP + H, H)]],
                              buf1, gsem1)

        g0.wait()
        pltpu.async_copy(buf0, out_hbm.at[pl.ds(base, H)], osem0)
        g1.wait()
        pltpu.async_copy(buf1, out_hbm.at[pl.ds(base + H, H)], osem1)
        return 0

    lax.fori_loop(0, SEQ_PER_W, seq_body, 0)

    last = wbase + (SEQ_PER_W - 1) * SP
    pltpu.make_async_copy(buf0, out_hbm.at[pl.ds(last, H)], osem0).wait()
    pltpu.make_async_copy(buf1, out_hbm.at[pl.ds(last + H, H)], osem1).wait()


def _add_body(g_ref, p_ref, o_ref):
    g = g_ref[...].reshape(BB, SP, D)
    p = p_ref[...]
    o_ref[...] = g[:, :S, :] + p[None, :S, :]


_add_kernel = pl.pallas_call(
    _add_body,
    grid=(B // BB,),
    in_specs=[
        pl.BlockSpec((BB * SP, D), lambda i: (i, 0)),
        pl.BlockSpec((SP, D), lambda i: (0, 0)),
    ],
    out_specs=pl.BlockSpec((BB, S, D), lambda i: (i, 0, 0)),
    out_shape=jax.ShapeDtypeStruct((B, S, D), jnp.float32),
)


def kernel(input_ids, position_ids, token_table, pos_table):
    ids = input_ids.astype(jnp.int32).reshape(B, S)
    ids_pad = jnp.pad(ids, ((0, 0), (0, SP - S))).reshape(-1)
    pids = jnp.pad(position_ids.astype(jnp.int32).reshape(-1), (0, SP - S))
    gathered, pos_eff = _gather_kernel(ids_pad, pids, token_table, pos_table)
    return _add_kernel(gathered, pos_eff)
